# baseline (speedup 1.0000x reference)
import jax
import jax.numpy as jnp
from jax.experimental import pallas as pl
from jax.experimental.pallas import tpu as pltpu
from jax.experimental.pallas import tpu_sc as plsc

_BLK = 16384
_LANES = 16


def _sc_body(x_vmem, pe_vmem, o_vmem):
    @plsc.parallel_loop(0, _BLK, step=_LANES, unroll=8)
    def _(c):
        slc = pl.ds(c, _LANES)
        o_vmem.at[slc][...] = x_vmem.at[slc][...] + pe_vmem.at[slc][...]


def kernel(x, pe_weight):
    B, S, D = x.shape
    xf = x.reshape(B * S * D)
    pef = pe_weight.reshape(S * D)
    n_pe_blocks = S * D // _BLK

    @pl.kernel(
        out_type=jax.ShapeDtypeStruct((B * S * D,), x.dtype),
        mesh=plsc.VectorSubcoreMesh(core_axis_name="c", subcore_axis_name="s"),
    )
    def run(x_hbm, pe_hbm, o_hbm):
        pltpu.emit_pipeline(
            _sc_body,
            grid=(B * S * D // _BLK,),
            in_specs=[
                pl.BlockSpec((_BLK,), lambda i: (i,)),
                pl.BlockSpec((_BLK,), lambda i: (i % n_pe_blocks,)),
            ],
            out_specs=[pl.BlockSpec((_BLK,), lambda i: (i,))],
            core_axis_name=("c", "s"),
            dimension_semantics=(pltpu.PARALLEL,),
        )(x_hbm, pe_hbm, o_hbm)

    return run(xf, pef).reshape(B, S, D)


# speedup vs baseline: 2.3390x; 2.3390x over previous
import jax
import jax.numpy as jnp
from jax.experimental import pallas as pl
from jax.experimental.pallas import tpu as pltpu
from jax.experimental.pallas import tpu_sc as plsc

_BR = 16
_LANES = 16


def _sc_body(x_vmem, pe_vmem, o_vmem):
    x2 = x_vmem.at[0]
    o2 = o_vmem.at[0]
    ncols = pe_vmem.shape[1]

    @pl.loop(0, _BR)
    def _(r):
        @plsc.parallel_loop(0, ncols, step=_LANES, unroll=8)
        def _(c):
            slc = (pl.ds(r, 1), pl.ds(c, _LANES))
            o2.at[*slc][...] = x2.at[*slc][...] + pe_vmem.at[*slc][...]


def kernel(x, pe_weight):
    B, S, D = x.shape

    @pl.kernel(
        out_type=jax.ShapeDtypeStruct((B, S, D), x.dtype),
        mesh=plsc.VectorSubcoreMesh(core_axis_name="c", subcore_axis_name="s"),
        compiler_params=pltpu.CompilerParams(use_tc_tiling_on_sc=True),
    )
    def run(x_hbm, pe_hbm, o_hbm):
        pltpu.emit_pipeline(
            _sc_body,
            grid=(B, S // _BR),
            in_specs=[
                pl.BlockSpec((1, _BR, D), lambda b, i: (b, i, 0)),
                pl.BlockSpec((_BR, D), lambda b, i: (i, 0)),
            ],
            out_specs=[pl.BlockSpec((1, _BR, D), lambda b, i: (b, i, 0))],
            core_axis_name=("c", "s"),
            dimension_semantics=(pltpu.PARALLEL, pltpu.PARALLEL),
        )(x_hbm, pe_hbm, o_hbm)

    return run(x, pe_weight)
